# Initial kernel scaffold; baseline (speedup 1.0000x reference)
import jax
import jax.numpy as jnp
from jax.experimental import pallas as pl


def kernel(input_ids, token_embed, pos_embed):
    raise NotImplementedError("write your pallas kernel here")


# speedup vs baseline: 1.1482x; 1.1482x over previous
import functools

import jax
import jax.numpy as jnp
from jax import lax
from jax.experimental import pallas as pl
from jax.experimental.pallas import tpu as pltpu
from jax.experimental.pallas import tpu_sc as plsc

NC = 2
NS = 16
NW = NC * NS
LANES = 16


@jax.jit
def _embed_sc(ids_flat, token_embed, pos_embed):
    BT = ids_flat.shape[0]
    V, D = token_embed.shape
    T = pos_embed.shape[0]
    B = BT // T
    t_per_w = T // NW
    CH = 32
    nch = t_per_w // CH
    vecs_per_row = D // LANES
    vecs = CH * vecs_per_row

    mesh = plsc.VectorSubcoreMesh(
        core_axis_name="c", subcore_axis_name="s", num_cores=NC,
        num_subcores=NS)

    @functools.partial(
        pl.kernel,
        out_type=jax.ShapeDtypeStruct((BT, D), jnp.float32),
        mesh=mesh,
        scratch_types=[
            pltpu.VMEM((B, t_per_w), jnp.int32),
            pltpu.VMEM((t_per_w, D), jnp.float32),
            pltpu.VMEM((CH, D), jnp.float32),
            pltpu.SemaphoreType.DMA,
        ],
    )
    def k(ids_hbm, tok_hbm, pos_hbm, out_hbm, idx_v, pos_v, buf, sem):
        wid = lax.axis_index("s") * NC + lax.axis_index("c")
        t0 = wid * t_per_w
        pltpu.sync_copy(pos_hbm.at[pl.ds(t0, t_per_w)], pos_v)
        for b in range(B):
            pltpu.sync_copy(ids_hbm.at[pl.ds(b * T + t0, t_per_w)],
                            idx_v.at[b])
            for h in range(nch):
                idx_slice = idx_v.at[b, pl.ds(h * CH, CH)]
                pltpu.async_copy(tok_hbm.at[idx_slice], buf, sem).wait()
                p_base = h * CH

                @plsc.parallel_loop(0, vecs, unroll=8)
                def _(i):
                    row = i // vecs_per_row
                    col = (i % vecs_per_row) * LANES
                    x = pos_v[p_base + row, pl.ds(col, LANES)]
                    plsc.addupdate(buf.at[row, pl.ds(col, LANES)], x)

                pltpu.sync_copy(
                    buf, out_hbm.at[pl.ds(b * T + t0 + h * CH, CH)])

    return k(ids_flat, token_embed, pos_embed)


def kernel(input_ids, token_embed, pos_embed):
    B, T = input_ids.shape
    D = token_embed.shape[1]
    ids_flat = input_ids.reshape(B * T).astype(jnp.int32)
    out = _embed_sc(ids_flat, token_embed, pos_embed[:T])
    return out.reshape(B, T, D)


# speedup vs baseline: 1.4260x; 1.2419x over previous
import functools

import jax
import jax.numpy as jnp
from jax import lax
from jax.experimental import pallas as pl
from jax.experimental.pallas import tpu as pltpu
from jax.experimental.pallas import tpu_sc as plsc

NC = 2
NS = 16
NW = NC * NS
LANES = 16


@jax.jit
def _embed_sc(ids_flat, token_embed, pos_embed):
    BT = ids_flat.shape[0]
    V, D = token_embed.shape
    T = pos_embed.shape[0]
    B = BT // T
    t_per_w = T // NW
    CH = 16
    nch = t_per_w // CH
    vecs_per_row = D // LANES
    vecs = CH * vecs_per_row

    mesh = plsc.VectorSubcoreMesh(
        core_axis_name="c", subcore_axis_name="s", num_cores=NC,
        num_subcores=NS)

    @functools.partial(
        pl.kernel,
        out_type=jax.ShapeDtypeStruct((BT, D), jnp.float32),
        mesh=mesh,
        scratch_types=[
            pltpu.VMEM((B, t_per_w), jnp.int32),
            pltpu.VMEM((t_per_w, D), jnp.float32),
            pltpu.VMEM((CH, D), jnp.float32),
            pltpu.VMEM((CH, D), jnp.float32),
            pltpu.SemaphoreType.DMA,
            pltpu.SemaphoreType.DMA,
            pltpu.SemaphoreType.DMA,
            pltpu.SemaphoreType.DMA,
        ],
    )
    def k(ids_hbm, tok_hbm, pos_hbm, out_hbm, idx_v, pos_v, buf0, buf1,
          gs0, gs1, ss0, ss1):
        wid = lax.axis_index("s") * NC + lax.axis_index("c")
        t0 = wid * t_per_w
        pltpu.sync_copy(pos_hbm.at[pl.ds(t0, t_per_w)], pos_v)
        for b in range(B):
            pltpu.sync_copy(ids_hbm.at[pl.ds(b * T + t0, t_per_w)],
                            idx_v.at[b])

        bufs = (buf0, buf1)
        gsem = (gs0, gs1)
        ssem = (ss0, ss1)
        chunks = [(b, h) for b in range(B) for h in range(nch)]
        n = len(chunks)
        gd = [None] * n
        sd = [None] * n

        def start_gather(i):
            b, h = chunks[i]
            idx_slice = idx_v.at[b, pl.ds(h * CH, CH)]
            gd[i] = pltpu.async_copy(tok_hbm.at[idx_slice], bufs[i % 2],
                                     gsem[i % 2])

        def add_pos(i):
            _, h = chunks[i]
            buf = bufs[i % 2]
            p_base = h * CH

            @plsc.parallel_loop(0, vecs, unroll=8)
            def _(v):
                row = v // vecs_per_row
                col = (v % vecs_per_row) * LANES
                x = pos_v[p_base + row, pl.ds(col, LANES)]
                plsc.addupdate(buf.at[row, pl.ds(col, LANES)], x)

        def start_store(i):
            b, h = chunks[i]
            sd[i] = pltpu.async_copy(
                bufs[i % 2], out_hbm.at[pl.ds(b * T + t0 + h * CH, CH)],
                ssem[i % 2])

        start_gather(0)
        for i in range(n):
            if i + 1 < n:
                if i >= 1:
                    sd[i - 1].wait()
                start_gather(i + 1)
            gd[i].wait()
            add_pos(i)
            start_store(i)
        sd[n - 2].wait()
        sd[n - 1].wait()

    return k(ids_flat, token_embed, pos_embed)


def kernel(input_ids, token_embed, pos_embed):
    B, T = input_ids.shape
    D = token_embed.shape[1]
    ids_flat = input_ids.reshape(B * T).astype(jnp.int32)
    out = _embed_sc(ids_flat, token_embed, pos_embed[:T])
    return out.reshape(B, T, D)


# speedup vs baseline: 1.5660x; 1.0982x over previous
import jax
import jax.numpy as jnp
from jax import lax
from jax.experimental import pallas as pl
from jax.experimental.pallas import tpu as pltpu
from jax.experimental.pallas import tpu_sc as plsc
import functools

NC = 2
NS = 16
NW = NC * NS
LANES = 16


@jax.jit
def _embed_sc(ids_w, token_embed, pos_embed):
    V, D = token_embed.shape
    T = pos_embed.shape[0]
    B = ids_w.shape[1] * NW // T
    t_per_w = T // NW
    CH = 16
    nch = t_per_w // CH
    vecs_per_row = D // LANES
    vecs = CH * vecs_per_row

    mesh = plsc.VectorSubcoreMesh(
        core_axis_name="c", subcore_axis_name="s", num_cores=NC,
        num_subcores=NS)

    @functools.partial(
        pl.kernel,
        out_type=jax.ShapeDtypeStruct((B * T, D), jnp.float32),
        mesh=mesh,
        scratch_types=[
            pltpu.VMEM((B * t_per_w,), jnp.int32),
            pltpu.VMEM((t_per_w, D), jnp.float32),
            pltpu.VMEM((CH, D), jnp.float32),
            pltpu.VMEM((CH, D), jnp.float32),
            pltpu.VMEM((CH, D), jnp.float32),
            pltpu.SemaphoreType.DMA,
            pltpu.SemaphoreType.DMA,
            pltpu.SemaphoreType.DMA,
            pltpu.SemaphoreType.DMA,
            pltpu.SemaphoreType.DMA,
            pltpu.SemaphoreType.DMA,
            pltpu.SemaphoreType.DMA,
            pltpu.SemaphoreType.DMA,
        ],
    )
    def k(ids_hbm, tok_hbm, pos_hbm, out_hbm, idx_v, pos_v, buf0, buf1,
          buf2, gs0, gs1, gs2, ss0, ss1, ss2, psem, isem):
        wid = lax.axis_index("s") * NC + lax.axis_index("c")
        t0 = wid * t_per_w
        idx_d = pltpu.async_copy(ids_hbm.at[wid], idx_v, isem)
        pos_d = pltpu.async_copy(pos_hbm.at[pl.ds(t0, t_per_w)], pos_v,
                                 psem)

        bufs = (buf0, buf1, buf2)
        gsem = (gs0, gs1, gs2)
        ssem = (ss0, ss1, ss2)
        chunks = [(b, h) for b in range(B) for h in range(nch)]
        n = len(chunks)
        gd = [None] * n
        sd = [None] * n

        def start_gather(i):
            b, h = chunks[i]
            idx_slice = idx_v.at[pl.ds((b * nch + h) * CH, CH)]
            gd[i] = pltpu.async_copy(tok_hbm.at[idx_slice], bufs[i % 3],
                                     gsem[i % 3])

        def add_pos(i):
            _, h = chunks[i]
            buf = bufs[i % 3]
            p_base = h * CH

            @plsc.parallel_loop(0, vecs, unroll=16)
            def _(v):
                row = v // vecs_per_row
                col = (v % vecs_per_row) * LANES
                x = pos_v[p_base + row, pl.ds(col, LANES)]
                plsc.addupdate(buf.at[row, pl.ds(col, LANES)], x)

        def start_store(i):
            b, h = chunks[i]
            sd[i] = pltpu.async_copy(
                bufs[i % 3], out_hbm.at[pl.ds(b * T + t0 + h * CH, CH)],
                ssem[i % 3])

        idx_d.wait()
        start_gather(0)
        start_gather(1)
        pos_d.wait()
        for i in range(n):
            if i + 2 < n:
                if i >= 1:
                    sd[i - 1].wait()
                start_gather(i + 2)
            gd[i].wait()
            add_pos(i)
            start_store(i)
        sd[n - 2].wait()
        sd[n - 1].wait()

    return k(ids_w, token_embed, pos_embed)


def kernel(input_ids, token_embed, pos_embed):
    B, T = input_ids.shape
    D = token_embed.shape[1]
    t_per_w = T // NW
    ids_w = (input_ids.astype(jnp.int32)
             .reshape(B, NW, t_per_w)
             .transpose(1, 0, 2)
             .reshape(NW, B * t_per_w))
    out = _embed_sc(ids_w, token_embed, pos_embed[:T])
    return out.reshape(B, T, D)
